# Initial kernel scaffold; baseline (speedup 1.0000x reference)
import jax
import jax.numpy as jnp
from jax.experimental import pallas as pl


def kernel(hidden_states, Wg, gw, pw, ow):
    raise NotImplementedError("write your pallas kernel here")


# speedup vs baseline: 1.0022x; 1.0022x over previous
import functools

import jax
import jax.numpy as jnp
from jax.experimental import pallas as pl
from jax.experimental.pallas import tpu as pltpu

E = 8
TOPK = 2
NEG = -1e30


def _router_body(x_ref, wgt_ref, logits_ref, w_ref, i0_ref, i1_ref,
                 w0_ref, w1_ref, c0_ref, call_ref, psum_ref, bl_ref):
    i = pl.program_id(0)
    nsteps = pl.num_programs(0)
    tb = x_ref.shape[0]

    lp = jnp.dot(x_ref[...], wgt_ref[...], preferred_element_type=jnp.float32)
    lanes = jax.lax.broadcasted_iota(jnp.int32, lp.shape, 1)
    valid = lanes < E
    l = jnp.where(valid, lp, NEG)

    m0 = jnp.max(l, axis=1, keepdims=True)
    i0 = jnp.min(jnp.where(l == m0, lanes, 127), axis=1, keepdims=True)
    l2 = jnp.where(lanes == i0, NEG, l)
    m1 = jnp.max(l2, axis=1, keepdims=True)
    i1 = jnp.min(jnp.where(l2 == m1, lanes, 127), axis=1, keepdims=True)

    w0 = jax.nn.sigmoid(m0 - m1)
    w1 = 1.0 - w0

    oh0 = (lanes == i0).astype(jnp.float32)
    oh1 = (lanes == i1).astype(jnp.float32)

    logits_ref[...] = lp[:, :E]
    w_ref[...] = (w0 * oh0 + w1 * oh1)[:, :E]
    i0_ref[...] = i0
    i1_ref[...] = i1
    w0_ref[...] = w0
    w1_ref[...] = w1

    p = jnp.where(valid, jnp.exp(l - m0), 0.0)
    p = p / jnp.sum(p, axis=1, keepdims=True)

    c0_part = jnp.sum(oh0, axis=0, keepdims=True)
    call_part = c0_part + jnp.sum(oh1, axis=0, keepdims=True)
    psum_part = jnp.sum(p, axis=0, keepdims=True)

    @pl.when(i == 0)
    def _init():
        c0_ref[...] = c0_part
        call_ref[...] = call_part
        psum_ref[...] = psum_part

    @pl.when(i > 0)
    def _acc():
        c0_ref[...] += c0_part
        call_ref[...] += call_part
        psum_ref[...] += psum_part

    @pl.when(i == nsteps - 1)
    def _fin():
        t_total = jnp.float32(nsteps * tb)
        bl = (jnp.float32(E) / (t_total * t_total)) * jnp.sum(
            call_ref[...] * psum_ref[...])
        bl_ref[...] = jnp.reshape(bl, (1, 1))


def _run_router(x, Wg):
    t, d = x.shape
    tb = 512 if t % 512 == 0 else t
    wgt = jnp.zeros((d, 128), jnp.float32).at[:, :E].set(Wg.T.astype(jnp.float32))
    grid = (t // tb,)
    outs = pl.pallas_call(
        _router_body,
        grid=grid,
        in_specs=[
            pl.BlockSpec((tb, d), lambda i: (i, 0)),
            pl.BlockSpec((d, 128), lambda i: (0, 0)),
        ],
        out_specs=[
            pl.BlockSpec((tb, E), lambda i: (i, 0)),
            pl.BlockSpec((tb, E), lambda i: (i, 0)),
            pl.BlockSpec((tb, 1), lambda i: (i, 0)),
            pl.BlockSpec((tb, 1), lambda i: (i, 0)),
            pl.BlockSpec((tb, 1), lambda i: (i, 0)),
            pl.BlockSpec((tb, 1), lambda i: (i, 0)),
            pl.BlockSpec((1, 128), lambda i: (0, 0)),
            pl.BlockSpec((1, 128), lambda i: (0, 0)),
            pl.BlockSpec((1, 128), lambda i: (0, 0)),
            pl.BlockSpec((1, 1), lambda i: (0, 0)),
        ],
        out_shape=[
            jax.ShapeDtypeStruct((t, E), jnp.float32),
            jax.ShapeDtypeStruct((t, E), jnp.float32),
            jax.ShapeDtypeStruct((t, 1), jnp.int32),
            jax.ShapeDtypeStruct((t, 1), jnp.int32),
            jax.ShapeDtypeStruct((t, 1), jnp.float32),
            jax.ShapeDtypeStruct((t, 1), jnp.float32),
            jax.ShapeDtypeStruct((1, 128), jnp.float32),
            jax.ShapeDtypeStruct((1, 128), jnp.float32),
            jax.ShapeDtypeStruct((1, 128), jnp.float32),
            jax.ShapeDtypeStruct((1, 1), jnp.float32),
        ],
    )(x, wgt)
    return outs


def _dense_body(x_ref, gw_ref, pw_ref, ow_ref, w_ref, out_ref):
    e = pl.program_id(1)
    f = pl.program_id(2)

    lanes = jax.lax.broadcasted_iota(jnp.int32, w_ref.shape, 1)
    wcol = jnp.sum(jnp.where(lanes == e, w_ref[...], 0.0), axis=1, keepdims=True)

    g = jnp.dot(x_ref[...], gw_ref[0], preferred_element_type=jnp.float32)
    p = jnp.dot(x_ref[...], pw_ref[0], preferred_element_type=jnp.float32)
    h = (g * (p * jax.nn.sigmoid(p))) * wcol
    part = jnp.dot(h, ow_ref[0], preferred_element_type=jnp.float32)

    @pl.when((e == 0) & (f == 0))
    def _init():
        out_ref[...] = part

    @pl.when((e > 0) | (f > 0))
    def _acc():
        out_ref[...] += part


def kernel(hidden_states, Wg, gw, pw, ow):
    b, s, d = hidden_states.shape
    x = hidden_states.reshape(-1, d).astype(jnp.float32)
    t = x.shape[0]
    ne, _, fdim = gw.shape

    (logits, wdense, _i0, _i1, _w0, _w1, _c0, _call, _psum, bl) = _run_router(x, Wg)

    tb = 512 if t % 512 == 0 else t
    fb = 512 if fdim % 512 == 0 else fdim
    grid = (t // tb, ne, fdim // fb)

    out = pl.pallas_call(
        _dense_body,
        grid=grid,
        in_specs=[
            pl.BlockSpec((tb, d), lambda i, e, f: (i, 0)),
            pl.BlockSpec((1, d, fb), lambda i, e, f: (e, 0, f)),
            pl.BlockSpec((1, d, fb), lambda i, e, f: (e, 0, f)),
            pl.BlockSpec((1, fb, d), lambda i, e, f: (e, f, 0)),
            pl.BlockSpec((tb, E), lambda i, e, f: (i, 0)),
        ],
        out_specs=pl.BlockSpec((tb, d), lambda i, e, f: (i, 0)),
        out_shape=jax.ShapeDtypeStruct((t, d), jnp.float32),
    )(x, gw.astype(jnp.float32), pw.astype(jnp.float32),
      ow.astype(jnp.float32), wdense)

    return (out.reshape(b, s, d), logits, bl[0, 0])
